# Initial kernel scaffold; baseline (speedup 1.0000x reference)
import jax
import jax.numpy as jnp
from jax.experimental import pallas as pl


def kernel(seqs, item_embed):
    raise NotImplementedError("write your pallas kernel here")


# speedup vs baseline: 2.4047x; 2.4047x over previous
import functools

import jax
import jax.numpy as jnp
from jax import lax
from jax.experimental import pallas as pl
from jax.experimental.pallas import tpu as pltpu
from jax.experimental.pallas import tpu_sc as plsc

ITEM_NUM = 1000000
EMBED_DIM = 32
SAMPLE_NUM = 256
BATCH = 1024
SEQ_LEN = 20
NSTEP = SEQ_LEN - 1

NC, NS, LANES = 2, 16, 16
NW = NC * NS
BPW = BATCH // NW
QPW = BPW * NSTEP
NQ = BATCH * NSTEP
HALF = SAMPLE_NUM // 2


def _splat(x):
    return jnp.broadcast_to(x, (LANES,))


def _rsqrt_vec(x):
    i = plsc.bitcast(x, jnp.int32)
    i = jnp.int32(0x5F3759DF) - lax.shift_right_logical(i, 1)
    y = plsc.bitcast(i, jnp.float32)
    for _ in range(3):
        y = y * (jnp.float32(1.5) - jnp.float32(0.5) * x * y * y)
    return y


def _sc_body(seqflat_hbm, table_hbm, samp_hbm, out_hbm,
             idxseq_v, rows_v, h_v, posv_v, mvec_v, svec_v,
             ids0_v, ids1_v, buf0_v, buf1_v, score_v,
             gsem, sem0, sem1):
    wid = lax.axis_index("s") * NC + lax.axis_index("c")
    base_b = wid * BPW
    iota16 = lax.iota(jnp.int32, LANES)

    pltpu.sync_copy(seqflat_hbm.at[pl.ds(base_b * SEQ_LEN, BPW * SEQ_LEN)],
                    idxseq_v)
    nseq = BPW * SEQ_LEN
    handles = []
    for c in range(nseq // 128):
        handles.append(pltpu.async_copy(
            table_hbm.at[idxseq_v.at[pl.ds(c * 128, 128)]],
            rows_v.at[pl.ds(c * 128, 128)], gsem))
    for h in handles:
        h.wait()

    def p1_body(bl, carry):
        r0 = bl * SEQ_LEN
        s0 = jnp.zeros((LANES,), jnp.float32)
        s1 = jnp.zeros((LANES,), jnp.float32)
        for j in range(SEQ_LEN):
            e0 = rows_v[r0 + j, pl.ds(0, 16)]
            e1 = rows_v[r0 + j, pl.ds(16, 16)]
            inv = _rsqrt_vec(_splat(jnp.sum(e0 * e0) + jnp.sum(e1 * e1)))
            en0 = e0 * inv
            en1 = e1 * inv
            if j >= 1:
                hinv = _rsqrt_vec(_splat(jnp.sum(s0 * s0) + jnp.sum(s1 * s1)))
                h0 = s0 * hinv
                h1 = s1 * hinv
                q = bl * NSTEP + (j - 1)
                h_v[pl.ds(q * EMBED_DIM, 16)] = h0
                h_v[pl.ds(q * EMBED_DIM + 16, 16)] = h1
                posv_v[pl.ds(q * LANES, LANES)] = en0 * h0 + en1 * h1
            s0 = s0 + en0
            s1 = s1 + en1
        return carry

    lax.fori_loop(0, BPW, p1_body, 0)

    samp_base = wid * QPW * SAMPLE_NUM

    def fire(q, ids_v, buf_v, sem):
        pltpu.sync_copy(
            samp_hbm.at[pl.ds(samp_base + q * SAMPLE_NUM, SAMPLE_NUM)], ids_v)
        pltpu.async_copy(table_hbm.at[ids_v.at[pl.ds(0, HALF)]],
                         buf_v.at[pl.ds(0, HALF)], sem)
        pltpu.async_copy(table_hbm.at[ids_v.at[pl.ds(HALF, HALF)]],
                         buf_v.at[pl.ds(HALF, HALF)], sem)

    def drain(buf_v, sem):
        pltpu.make_async_copy(table_hbm.at[pl.ds(0, SAMPLE_NUM)],
                              buf_v, sem).wait()

    def compute(q, buf_v):
        h0 = h_v[pl.ds(q * EMBED_DIM, 16)]
        h1 = h_v[pl.ds(q * EMBED_DIM + 16, 16)]
        hs = [h0[d] for d in range(16)] + [h1[d] for d in range(16)]

        def chunk_body(c, mx):
            rows = iota16 + c * LANES
            acc_d = jnp.zeros((LANES,), jnp.float32)
            acc_s = jnp.zeros((LANES,), jnp.float32)
            for d in range(EMBED_DIM):
                g = plsc.load_gather(buf_v, [rows, _splat(jnp.int32(d))])
                acc_d = acc_d + g * _splat(hs[d])
                acc_s = acc_s + g * g
            s = acc_d * _rsqrt_vec(acc_s)
            score_v[pl.ds(c * LANES, LANES)] = s
            return jnp.maximum(mx, s)

        mx = lax.fori_loop(0, SAMPLE_NUM // LANES, chunk_body,
                           jnp.full((LANES,), -2.0, jnp.float32))

        def sum_body(c, acc):
            return acc + jnp.exp(score_v[pl.ds(c * LANES, LANES)] - mx)

        sacc = lax.fori_loop(0, SAMPLE_NUM // LANES, sum_body,
                             jnp.zeros((LANES,), jnp.float32))
        mvec_v[pl.ds(q * LANES, LANES)] = mx
        svec_v[pl.ds(q * LANES, LANES)] = sacc

    fire(0, ids0_v, buf0_v, sem0)

    def pair_body(p, carry):
        q0 = 2 * p
        fire(q0 + 1, ids1_v, buf1_v, sem1)
        drain(buf0_v, sem0)
        compute(q0, buf0_v)

        @pl.when(q0 + 2 < QPW)
        def _():
            fire(q0 + 2, ids0_v, buf0_v, sem0)

        drain(buf1_v, sem1)
        compute(q0 + 1, buf1_v)
        return carry

    lax.fori_loop(0, QPW // 2, pair_body, 0)

    qv = wid * QPW * LANES
    pltpu.sync_copy(mvec_v, out_hbm.at[pl.ds(0 * NQ * LANES + qv, QPW * LANES)])
    pltpu.sync_copy(svec_v, out_hbm.at[pl.ds(1 * NQ * LANES + qv, QPW * LANES)])
    pltpu.sync_copy(posv_v, out_hbm.at[pl.ds(2 * NQ * LANES + qv, QPW * LANES)])


_sc_kernel = functools.partial(
    pl.kernel,
    mesh=plsc.VectorSubcoreMesh(core_axis_name="c", subcore_axis_name="s"),
    compiler_params=pltpu.CompilerParams(needs_layout_passes=False,
                                         use_tc_tiling_on_sc=False),
    out_type=jax.ShapeDtypeStruct((3 * NQ * LANES,), jnp.float32),
    scratch_types=[
        pltpu.VMEM((BPW * SEQ_LEN,), jnp.int32),
        pltpu.VMEM((BPW * SEQ_LEN, EMBED_DIM), jnp.float32),
        pltpu.VMEM((QPW * EMBED_DIM,), jnp.float32),
        pltpu.VMEM((QPW * LANES,), jnp.float32),
        pltpu.VMEM((QPW * LANES,), jnp.float32),
        pltpu.VMEM((QPW * LANES,), jnp.float32),
        pltpu.VMEM((SAMPLE_NUM,), jnp.int32),
        pltpu.VMEM((SAMPLE_NUM,), jnp.int32),
        pltpu.VMEM((SAMPLE_NUM, EMBED_DIM), jnp.float32),
        pltpu.VMEM((SAMPLE_NUM, EMBED_DIM), jnp.float32),
        pltpu.VMEM((SAMPLE_NUM,), jnp.float32),
        pltpu.SemaphoreType.DMA,
        pltpu.SemaphoreType.DMA,
        pltpu.SemaphoreType.DMA,
    ],
)(_sc_body)


def _tc_finish(x_ref, o_ref):
    m = x_ref[0]
    s = x_ref[1]
    pv = x_ref[2]
    big = jnp.max(m, axis=0, keepdims=True)
    se = jnp.sum(s * jnp.exp(m - big), axis=0)
    loss = jnp.sum(big[0] + jnp.log(se)) - jnp.sum(pv)
    o_ref[...] = jnp.reshape(loss, (1, 1))


def kernel(seqs, item_embed):
    B, L = seqs.shape
    neg_key = jax.random.key(1234)
    samp = jnp.stack(
        [jax.random.randint(jax.random.fold_in(neg_key, i), (B, SAMPLE_NUM),
                            0, ITEM_NUM, dtype=jnp.int32)
         for i in range(1, L)], axis=1)
    out = _sc_kernel(seqs.reshape(-1), item_embed, samp.reshape(-1))
    x = jnp.transpose(out.reshape(3, NQ, LANES), (0, 2, 1))
    loss = pl.pallas_call(
        _tc_finish,
        out_shape=jax.ShapeDtypeStruct((1, 1), jnp.float32),
    )(x)
    return loss[0, 0]


# speedup vs baseline: 2.6180x; 1.0887x over previous
import functools

import jax
import jax.numpy as jnp
from jax import lax
from jax.experimental import pallas as pl
from jax.experimental.pallas import tpu as pltpu
from jax.experimental.pallas import tpu_sc as plsc

ITEM_NUM = 1000000
EMBED_DIM = 32
SAMPLE_NUM = 256
BATCH = 1024
SEQ_LEN = 20
NSTEP = SEQ_LEN - 1

NC, NS, LANES = 2, 16, 16
NW = NC * NS
BPW = BATCH // NW
QPW = BPW * NSTEP
NQ = BATCH * NSTEP
G = 4
GROWS = G * SAMPLE_NUM
NGRP = QPW // G
BG = BPW // G


def _splat(x):
    return jnp.broadcast_to(x, (LANES,))


def _rsqrt_vec(x):
    i = plsc.bitcast(x, jnp.int32)
    i = jnp.int32(0x5F3759DF) - lax.shift_right_logical(i, 1)
    y = plsc.bitcast(i, jnp.float32)
    for _ in range(3):
        y = y * (jnp.float32(1.5) - jnp.float32(0.5) * x * y * y)
    return y


def _sc_body(seqflat_hbm, table_hbm, samp_hbm, out_hbm,
             idxseq_v, rows_v, h_v, posv_v, score_v,
             idsA_v, idsB_v, bufA_v, bufB_v, stageA_v, stageB_v,
             gsem, semA, semB, isemA, isemB, osemA, osemB):
    wid = lax.axis_index("s") * NC + lax.axis_index("c")
    base_b = wid * BPW
    iota16 = lax.iota(jnp.int32, LANES)

    pltpu.sync_copy(seqflat_hbm.at[pl.ds(base_b * SEQ_LEN, BPW * SEQ_LEN)],
                    idxseq_v)
    nseq = BPW * SEQ_LEN
    handles = []
    for c in range(nseq // 128):
        handles.append(pltpu.async_copy(
            table_hbm.at[idxseq_v.at[pl.ds(c * 128, 128)]],
            rows_v.at[pl.ds(c * 128, 128)], gsem))
    for h in handles:
        h.wait()

    def p1_body(bl, carry):
        r0 = bl * SEQ_LEN
        s0 = jnp.zeros((LANES,), jnp.float32)
        s1 = jnp.zeros((LANES,), jnp.float32)
        for j in range(SEQ_LEN):
            e0 = rows_v[r0 + j, pl.ds(0, 16)]
            e1 = rows_v[r0 + j, pl.ds(16, 16)]
            inv = _rsqrt_vec(_splat(jnp.sum(e0 * e0) + jnp.sum(e1 * e1)))
            en0 = e0 * inv
            en1 = e1 * inv
            if j >= 1:
                hinv = _rsqrt_vec(_splat(jnp.sum(s0 * s0) + jnp.sum(s1 * s1)))
                h0 = s0 * hinv
                h1 = s1 * hinv
                q = bl * NSTEP + (j - 1)
                h_v[pl.ds(q * EMBED_DIM, 16)] = h0
                h_v[pl.ds(q * EMBED_DIM + 16, 16)] = h1
                posv_v[pl.ds(q * LANES, LANES)] = en0 * h0 + en1 * h1
            s0 = s0 + en0
            s1 = s1 + en1
        return carry

    lax.fori_loop(0, BPW, p1_body, 0)


    def samp_off(g):
        i = g // BG
        bg = g % BG
        return pl.multiple_of((i * BATCH + base_b + bg * G) * SAMPLE_NUM, 256)

    def start_ids(g, ids_v, isem):
        pltpu.async_copy(samp_hbm.at[pl.ds(samp_off(g), GROWS)], ids_v, isem)

    def wait_ids(ids_v, isem):
        pltpu.make_async_copy(samp_hbm.at[pl.ds(0, GROWS)], ids_v, isem).wait()

    def fire_rows(ids_v, buf_v, sem):
        for k in range(GROWS // 128):
            pltpu.async_copy(table_hbm.at[ids_v.at[pl.ds(k * 128, 128)]],
                             buf_v.at[pl.ds(k * 128, 128)], sem)

    def wait_rows(buf_v, sem):
        pltpu.make_async_copy(table_hbm.at[pl.ds(0, GROWS)], buf_v, sem).wait()

    def drain_stage(stage_v, osem):
        pltpu.make_async_copy(stage_v, out_hbm.at[pl.ds(0, G * 2 * LANES)],
                              osem).wait()

    def compute_group(g, buf_v, stage_v):
        i = g // BG
        bg = g % BG
        for qi in range(G):
            hrow = ((bg * G + qi) * NSTEP + i) * EMBED_DIM
            h0 = h_v[pl.ds(hrow, 16)]
            h1 = h_v[pl.ds(hrow + 16, 16)]
            hs = [h0[d] for d in range(16)] + [h1[d] for d in range(16)]
            rbase = qi * SAMPLE_NUM

            def chunk_body(c, mx):
                rows = iota16 + (rbase + c * LANES)
                acc_d = jnp.zeros((LANES,), jnp.float32)
                acc_s = jnp.zeros((LANES,), jnp.float32)
                for d in range(EMBED_DIM):
                    g_ = plsc.load_gather(buf_v, [rows, _splat(jnp.int32(d))])
                    acc_d = acc_d + g_ * _splat(hs[d])
                    acc_s = acc_s + g_ * g_
                s = acc_d * _rsqrt_vec(acc_s)
                score_v[pl.ds(c * LANES, LANES)] = s
                return jnp.maximum(mx, s)

            mx = lax.fori_loop(0, SAMPLE_NUM // LANES, chunk_body,
                               jnp.full((LANES,), -2.0, jnp.float32))

            def sum_body(c, acc):
                return acc + jnp.exp(score_v[pl.ds(c * LANES, LANES)] - mx)

            sacc = lax.fori_loop(0, SAMPLE_NUM // LANES, sum_body,
                                 jnp.zeros((LANES,), jnp.float32))
            stage_v[pl.ds(qi * 2 * LANES, LANES)] = mx
            stage_v[pl.ds(qi * 2 * LANES + LANES, LANES)] = sacc

    def fire_stage(g, stage_v, osem):
        pltpu.async_copy(stage_v,
                         out_hbm.at[pl.ds(pl.multiple_of(samp_off(g) // 8, 32),
                                          G * 2 * LANES)],
                         osem)


    start_ids(0, idsA_v, isemA)
    start_ids(1, idsB_v, isemB)
    wait_ids(idsA_v, isemA)
    fire_rows(idsA_v, bufA_v, semA)
    start_ids(2, idsA_v, isemA)
    wait_ids(idsB_v, isemB)
    fire_rows(idsB_v, bufB_v, semB)
    start_ids(3, idsB_v, isemB)

    def pipe_body(t, carry):
        gA = 2 * t
        gB = 2 * t + 1

        @pl.when(t > 0)
        def _():
            drain_stage(stageA_v, osemA)
        wait_rows(bufA_v, semA)
        compute_group(gA, bufA_v, stageA_v)
        fire_stage(gA, stageA_v, osemA)

        @pl.when(gA + 2 < NGRP)
        def _():
            wait_ids(idsA_v, isemA)
            fire_rows(idsA_v, bufA_v, semA)

        @pl.when(gA + 4 < NGRP)
        def _():
            start_ids(gA + 4, idsA_v, isemA)

        @pl.when(t > 0)
        def _():
            drain_stage(stageB_v, osemB)
        wait_rows(bufB_v, semB)
        compute_group(gB, bufB_v, stageB_v)
        fire_stage(gB, stageB_v, osemB)

        @pl.when(gB + 2 < NGRP)
        def _():
            wait_ids(idsB_v, isemB)
            fire_rows(idsB_v, bufB_v, semB)

        @pl.when(gB + 4 < NGRP)
        def _():
            start_ids(gB + 4, idsB_v, isemB)
        return carry

    lax.fori_loop(0, NGRP // 2, pipe_body, 0)
    drain_stage(stageA_v, osemA)
    drain_stage(stageB_v, osemB)

    pltpu.sync_copy(posv_v,
                    out_hbm.at[pl.ds(NQ * 2 * LANES + wid * QPW * LANES,
                                     QPW * LANES)])


_sc_kernel = functools.partial(
    pl.kernel,
    mesh=plsc.VectorSubcoreMesh(core_axis_name="c", subcore_axis_name="s"),
    compiler_params=pltpu.CompilerParams(needs_layout_passes=False,
                                         use_tc_tiling_on_sc=False),
    out_type=jax.ShapeDtypeStruct((3 * NQ * LANES,), jnp.float32),
    scratch_types=[
        pltpu.VMEM((BPW * SEQ_LEN,), jnp.int32),
        pltpu.VMEM((BPW * SEQ_LEN, EMBED_DIM), jnp.float32),
        pltpu.VMEM((QPW * EMBED_DIM,), jnp.float32),
        pltpu.VMEM((QPW * LANES,), jnp.float32),
        pltpu.VMEM((SAMPLE_NUM,), jnp.float32),
        pltpu.VMEM((GROWS,), jnp.int32),
        pltpu.VMEM((GROWS,), jnp.int32),
        pltpu.VMEM((GROWS, EMBED_DIM), jnp.float32),
        pltpu.VMEM((GROWS, EMBED_DIM), jnp.float32),
        pltpu.VMEM((G * 2 * LANES,), jnp.float32),
        pltpu.VMEM((G * 2 * LANES,), jnp.float32),
        pltpu.SemaphoreType.DMA,
        pltpu.SemaphoreType.DMA,
        pltpu.SemaphoreType.DMA,
        pltpu.SemaphoreType.DMA,
        pltpu.SemaphoreType.DMA,
        pltpu.SemaphoreType.DMA,
        pltpu.SemaphoreType.DMA,
    ],
)(_sc_body)


def _tc_finish(x_ref, o_ref):
    m = x_ref[0]
    s = x_ref[1]
    pv = x_ref[2]
    big = jnp.max(m, axis=0, keepdims=True)
    se = jnp.sum(s * jnp.exp(m - big), axis=0)
    loss = jnp.sum(big[0] + jnp.log(se)) - jnp.sum(pv)
    o_ref[...] = jnp.reshape(loss, (1, 1))


def kernel(seqs, item_embed):
    B, L = seqs.shape
    neg_key = jax.random.key(1234)
    samp = jnp.stack(
        [jax.random.randint(jax.random.fold_in(neg_key, i), (B, SAMPLE_NUM),
                            0, ITEM_NUM, dtype=jnp.int32)
         for i in range(1, L)], axis=0)
    out = _sc_kernel(seqs.reshape(-1), item_embed, samp.reshape(-1))
    ms = out[:NQ * 2 * LANES].reshape(NQ, 2, LANES)
    pv = out[NQ * 2 * LANES:].reshape(NQ, LANES)
    x = jnp.stack([ms[:, 0, :].T, ms[:, 1, :].T, pv.T])
    loss = pl.pallas_call(
        _tc_finish,
        out_shape=jax.ShapeDtypeStruct((1, 1), jnp.float32),
    )(x)
    return loss[0, 0]


# speedup vs baseline: 5.0685x; 1.9360x over previous
import functools

import jax
import jax.numpy as jnp
from jax import lax
from jax.experimental import pallas as pl
from jax.experimental.pallas import tpu as pltpu
from jax.experimental.pallas import tpu_sc as plsc

ITEM_NUM = 1000000
EMBED_DIM = 32
SAMPLE_NUM = 256
BATCH = 1024
SEQ_LEN = 20
NSTEP = SEQ_LEN - 1

NC, NS, LANES = 2, 16, 16
NW = NC * NS
BPW = BATCH // NW
QPW = BPW * NSTEP
NQ = BATCH * NSTEP
G = 4
GROWS = G * SAMPLE_NUM
NGRP = QPW // G
BG = BPW // G


def _splat(x):
    return jnp.broadcast_to(x, (LANES,))


def _rsqrt_vec(x):
    i = plsc.bitcast(x, jnp.int32)
    i = jnp.int32(0x5F3759DF) - lax.shift_right_logical(i, 1)
    y = plsc.bitcast(i, jnp.float32)
    for _ in range(3):
        y = y * (jnp.float32(1.5) - jnp.float32(0.5) * x * y * y)
    return y


def _sc_body(seqflat_hbm, table_hbm, samp_hbm, out_hbm,
             idxseq_v, rows_v, h_v, posv_v, score_v, hh_v,
             idsA_v, idsB_v, bufA_v, bufB_v, stageA_v, stageB_v,
             gsem, semA, semB, isemA, isemB, osemA, osemB):
    wid = lax.axis_index("s") * NC + lax.axis_index("c")
    base_b = wid * BPW
    iota16 = lax.iota(jnp.int32, LANES)

    pltpu.sync_copy(seqflat_hbm.at[pl.ds(base_b * SEQ_LEN, BPW * SEQ_LEN)],
                    idxseq_v)
    nseq = BPW * SEQ_LEN
    handles = []
    for c in range(nseq // 128):
        handles.append(pltpu.async_copy(
            table_hbm.at[idxseq_v.at[pl.ds(c * 128, 128)]],
            rows_v.at[pl.ds(c * 128, 128)], gsem))
    for h in handles:
        h.wait()

    def p1_body(bl, carry):
        r0 = bl * SEQ_LEN
        s0 = jnp.zeros((LANES,), jnp.float32)
        s1 = jnp.zeros((LANES,), jnp.float32)
        for j in range(SEQ_LEN):
            e0 = rows_v[r0 + j, pl.ds(0, 16)]
            e1 = rows_v[r0 + j, pl.ds(16, 16)]
            inv = _rsqrt_vec(_splat(jnp.sum(e0 * e0) + jnp.sum(e1 * e1)))
            en0 = e0 * inv
            en1 = e1 * inv
            if j >= 1:
                hinv = _rsqrt_vec(_splat(jnp.sum(s0 * s0) + jnp.sum(s1 * s1)))
                h0 = s0 * hinv
                h1 = s1 * hinv
                q = bl * NSTEP + (j - 1)
                h_v[pl.ds(q * EMBED_DIM, 16)] = h0
                h_v[pl.ds(q * EMBED_DIM + 16, 16)] = h1
                posv_v[pl.ds(q * LANES, LANES)] = en0 * h0 + en1 * h1
            s0 = s0 + en0
            s1 = s1 + en1
        return carry

    lax.fori_loop(0, BPW, p1_body, 0)


    def samp_off(g):
        i = g // BG
        bg = g % BG
        return pl.multiple_of((i * BATCH + base_b + bg * G) * SAMPLE_NUM, 256)

    def start_ids(g, ids_v, isem):
        pltpu.async_copy(samp_hbm.at[pl.ds(samp_off(g), GROWS)], ids_v, isem)

    def wait_ids(ids_v, isem):
        pltpu.make_async_copy(samp_hbm.at[pl.ds(0, GROWS)], ids_v, isem).wait()

    def fire_rows(ids_v, buf_v, sem):
        for k in range(GROWS // 128):
            pltpu.async_copy(table_hbm.at[ids_v.at[pl.ds(k * 128, 128)]],
                             buf_v.at[pl.ds(k * 128, 128)], sem)

    def wait_rows(buf_v, sem):
        pltpu.make_async_copy(table_hbm.at[pl.ds(0, GROWS)], buf_v, sem).wait()

    def drain_stage(stage_v, osem):
        pltpu.make_async_copy(stage_v, out_hbm.at[pl.ds(0, G * 2 * LANES)],
                              osem).wait()

    def compute_group(g, buf_v, stage_v):
        i = g // BG
        bg = g % BG
        for qi in range(G):
            hrow = ((bg * G + qi) * NSTEP + i) * EMBED_DIM
            h0 = h_v[pl.ds(hrow, 16)]
            h1 = h_v[pl.ds(hrow + 16, 16)]
            hh_v[pl.ds(0, 16)] = h0
            hh_v[pl.ds(16, 16)] = h1
            hh_v[pl.ds(32, 16)] = h0
            hh_v[pl.ds(48, 16)] = h1
            rbase = qi * SAMPLE_NUM

            def chunk_body(c, mx):
                rows = iota16 + (rbase + c * LANES)
                acc_d = jnp.zeros((LANES,), jnp.float32)
                acc_s = jnp.zeros((LANES,), jnp.float32)
                diag = iota16
                for k in range(EMBED_DIM):
                    g_ = plsc.load_gather(buf_v, [rows, diag])
                    acc_d = acc_d + g_ * hh_v[pl.ds(k, 16)]
                    acc_s = acc_s + g_ * g_
                    diag = jnp.bitwise_and(diag + 1, EMBED_DIM - 1)
                s = acc_d * _rsqrt_vec(acc_s)
                score_v[pl.ds(c * LANES, LANES)] = s
                return jnp.maximum(mx, s)

            mx = lax.fori_loop(0, SAMPLE_NUM // LANES, chunk_body,
                               jnp.full((LANES,), -2.0, jnp.float32))

            def sum_body(c, acc):
                return acc + jnp.exp(score_v[pl.ds(c * LANES, LANES)] - mx)

            sacc = lax.fori_loop(0, SAMPLE_NUM // LANES, sum_body,
                                 jnp.zeros((LANES,), jnp.float32))
            stage_v[pl.ds(qi * 2 * LANES, LANES)] = mx
            stage_v[pl.ds(qi * 2 * LANES + LANES, LANES)] = sacc

    def fire_stage(g, stage_v, osem):
        pltpu.async_copy(stage_v,
                         out_hbm.at[pl.ds(pl.multiple_of(samp_off(g) // 8, 32),
                                          G * 2 * LANES)],
                         osem)


    start_ids(0, idsA_v, isemA)
    start_ids(1, idsB_v, isemB)
    wait_ids(idsA_v, isemA)
    fire_rows(idsA_v, bufA_v, semA)
    start_ids(2, idsA_v, isemA)
    wait_ids(idsB_v, isemB)
    fire_rows(idsB_v, bufB_v, semB)
    start_ids(3, idsB_v, isemB)

    def pipe_body(t, carry):
        gA = 2 * t
        gB = 2 * t + 1

        @pl.when(t > 0)
        def _():
            drain_stage(stageA_v, osemA)
        wait_rows(bufA_v, semA)
        compute_group(gA, bufA_v, stageA_v)
        fire_stage(gA, stageA_v, osemA)

        @pl.when(gA + 2 < NGRP)
        def _():
            wait_ids(idsA_v, isemA)
            fire_rows(idsA_v, bufA_v, semA)

        @pl.when(gA + 4 < NGRP)
        def _():
            start_ids(gA + 4, idsA_v, isemA)

        @pl.when(t > 0)
        def _():
            drain_stage(stageB_v, osemB)
        wait_rows(bufB_v, semB)
        compute_group(gB, bufB_v, stageB_v)
        fire_stage(gB, stageB_v, osemB)

        @pl.when(gB + 2 < NGRP)
        def _():
            wait_ids(idsB_v, isemB)
            fire_rows(idsB_v, bufB_v, semB)

        @pl.when(gB + 4 < NGRP)
        def _():
            start_ids(gB + 4, idsB_v, isemB)
        return carry

    lax.fori_loop(0, NGRP // 2, pipe_body, 0)
    drain_stage(stageA_v, osemA)
    drain_stage(stageB_v, osemB)

    pltpu.sync_copy(posv_v,
                    out_hbm.at[pl.ds(NQ * 2 * LANES + wid * QPW * LANES,
                                     QPW * LANES)])


_sc_kernel = functools.partial(
    pl.kernel,
    mesh=plsc.VectorSubcoreMesh(core_axis_name="c", subcore_axis_name="s"),
    compiler_params=pltpu.CompilerParams(needs_layout_passes=False,
                                         use_tc_tiling_on_sc=False),
    out_type=jax.ShapeDtypeStruct((3 * NQ * LANES,), jnp.float32),
    scratch_types=[
        pltpu.VMEM((BPW * SEQ_LEN,), jnp.int32),
        pltpu.VMEM((BPW * SEQ_LEN, EMBED_DIM), jnp.float32),
        pltpu.VMEM((QPW * EMBED_DIM,), jnp.float32),
        pltpu.VMEM((QPW * LANES,), jnp.float32),
        pltpu.VMEM((SAMPLE_NUM,), jnp.float32),
        pltpu.VMEM((4 * LANES,), jnp.float32),
        pltpu.VMEM((GROWS,), jnp.int32),
        pltpu.VMEM((GROWS,), jnp.int32),
        pltpu.VMEM((GROWS, EMBED_DIM), jnp.float32),
        pltpu.VMEM((GROWS, EMBED_DIM), jnp.float32),
        pltpu.VMEM((G * 2 * LANES,), jnp.float32),
        pltpu.VMEM((G * 2 * LANES,), jnp.float32),
        pltpu.SemaphoreType.DMA,
        pltpu.SemaphoreType.DMA,
        pltpu.SemaphoreType.DMA,
        pltpu.SemaphoreType.DMA,
        pltpu.SemaphoreType.DMA,
        pltpu.SemaphoreType.DMA,
        pltpu.SemaphoreType.DMA,
    ],
)(_sc_body)


def _tc_finish(x_ref, o_ref):
    m = x_ref[0]
    s = x_ref[1]
    pv = x_ref[2]
    big = jnp.max(m, axis=0, keepdims=True)
    se = jnp.sum(s * jnp.exp(m - big), axis=0)
    loss = jnp.sum(big[0] + jnp.log(se)) - jnp.sum(pv)
    o_ref[...] = jnp.reshape(loss, (1, 1))


def kernel(seqs, item_embed):
    B, L = seqs.shape
    neg_key = jax.random.key(1234)
    samp = jnp.stack(
        [jax.random.randint(jax.random.fold_in(neg_key, i), (B, SAMPLE_NUM),
                            0, ITEM_NUM, dtype=jnp.int32)
         for i in range(1, L)], axis=0)
    out = _sc_kernel(seqs.reshape(-1), item_embed, samp.reshape(-1))
    ms = out[:NQ * 2 * LANES].reshape(NQ, 2, LANES)
    pv = out[NQ * 2 * LANES:].reshape(NQ, LANES)
    x = jnp.stack([ms[:, 0, :].T, ms[:, 1, :].T, pv.T])
    loss = pl.pallas_call(
        _tc_finish,
        out_shape=jax.ShapeDtypeStruct((1, 1), jnp.float32),
    )(x)
    return loss[0, 0]


# speedup vs baseline: 7.2039x; 1.4213x over previous
import functools

import jax
import jax.numpy as jnp
from jax import lax
from jax.experimental import pallas as pl
from jax.experimental.pallas import tpu as pltpu
from jax.experimental.pallas import tpu_sc as plsc

ITEM_NUM = 1000000
EMBED_DIM = 32
SAMPLE_NUM = 256
BATCH = 1024
SEQ_LEN = 20
NSTEP = SEQ_LEN - 1
NC, NS, LANES = 2, 16, 16
NW = NC * NS
BPW = BATCH // NW
QPW = BPW * NSTEP
NQ = BATCH * NSTEP
G = 4
GROWS = G * SAMPLE_NUM
NGRP = QPW // G
BG = BPW // G
OUTLEN = NQ + NW * LANES


def _splat(x):
    return jnp.broadcast_to(x, (LANES,))


def _rsqrt_vec(x):
    i = plsc.bitcast(x, jnp.int32)
    i = jnp.int32(0x5F3759DF) - lax.shift_right_logical(i, 1)
    y = plsc.bitcast(i, jnp.float32)
    for _ in range(3):
        y = y * (jnp.float32(1.5) - jnp.float32(0.5) * x * y * y)
    return y


def _sc_body(seqflat_hbm, table_hbm, samp_hbm, out_hbm,
             idxseq_v, rows_v, h_v, se_v, hh_v, pos_v,
             idsA_v, idsB_v, bufA_v, bufB_v,
             gsem, semA, semB, isemA, isemB):
    wid = lax.axis_index("s") * NC + lax.axis_index("c")
    base_b = wid * BPW
    iota16 = lax.iota(jnp.int32, LANES)

    pltpu.sync_copy(seqflat_hbm.at[pl.ds(base_b * SEQ_LEN, BPW * SEQ_LEN)],
                    idxseq_v)
    nseq = BPW * SEQ_LEN
    handles = []
    for c in range(nseq // 128):
        handles.append(pltpu.async_copy(
            table_hbm.at[idxseq_v.at[pl.ds(c * 128, 128)]],
            rows_v.at[pl.ds(c * 128, 128)], gsem))
    for h in handles:
        h.wait()

    def p1_body(bl, posacc):
        r0 = bl * SEQ_LEN
        s0 = jnp.zeros((LANES,), jnp.float32)
        s1 = jnp.zeros((LANES,), jnp.float32)
        for j in range(SEQ_LEN):
            e0 = rows_v[r0 + j, pl.ds(0, 16)]
            e1 = rows_v[r0 + j, pl.ds(16, 16)]
            inv = _rsqrt_vec(_splat(jnp.sum(e0 * e0) + jnp.sum(e1 * e1)))
            en0 = e0 * inv
            en1 = e1 * inv
            if j >= 1:
                hinv = _rsqrt_vec(_splat(jnp.sum(s0 * s0) + jnp.sum(s1 * s1)))
                h0 = s0 * hinv
                h1 = s1 * hinv
                q = bl * NSTEP + (j - 1)
                h_v[pl.ds(q * EMBED_DIM, 16)] = h0
                h_v[pl.ds(q * EMBED_DIM + 16, 16)] = h1
                posacc = posacc + jnp.sum(en0 * h0 + en1 * h1)
            s0 = s0 + en0
            s1 = s1 + en1
        return posacc

    posacc = lax.fori_loop(0, BPW, p1_body, jnp.float32(0.0))
    pos_v[...] = _splat(posacc)


    def start_ids(g, ids_v, isem):
        i = g // BG
        bg = g % BG
        col = pl.multiple_of((base_b + bg * G) * SAMPLE_NUM, 256)
        pltpu.async_copy(samp_hbm.at[i, pl.ds(col, GROWS)], ids_v, isem)

    def wait_ids(ids_v, isem):
        pltpu.make_async_copy(samp_hbm.at[0, pl.ds(0, GROWS)], ids_v,
                              isem).wait()

    def fire_rows(ids_v, buf_v, sem):
        for k in range(GROWS // 128):
            pltpu.async_copy(table_hbm.at[ids_v.at[pl.ds(k * 128, 128)]],
                             buf_v.at[pl.ds(k * 128, 128)], sem)

    def wait_rows(buf_v, sem):
        pltpu.make_async_copy(table_hbm.at[pl.ds(0, GROWS)], buf_v, sem).wait()

    def compute_group(g, buf_v, pack):
        i = g // BG
        bg = g % BG
        for qi in range(G):
            hrow = ((bg * G + qi) * NSTEP + i) * EMBED_DIM
            h0 = h_v[pl.ds(hrow, 16)]
            h1 = h_v[pl.ds(hrow + 16, 16)]
            hh_v[pl.ds(0, 16)] = h0
            hh_v[pl.ds(16, 16)] = h1
            hh_v[pl.ds(32, 16)] = h0
            hh_v[pl.ds(48, 16)] = h1
            rbase = qi * SAMPLE_NUM

            def chunk_body(c, acc):
                rows = iota16 + (rbase + c * LANES)
                acc_d = jnp.zeros((LANES,), jnp.float32)
                acc_s = jnp.zeros((LANES,), jnp.float32)
                diag = iota16
                for k in range(EMBED_DIM):
                    g_ = plsc.load_gather(buf_v, [rows, diag])
                    acc_d = acc_d + g_ * hh_v[pl.ds(k, 16)]
                    acc_s = acc_s + g_ * g_
                    diag = jnp.bitwise_and(diag + 1, EMBED_DIM - 1)
                return acc + jnp.exp(acc_d * _rsqrt_vec(acc_s)
                                     - jnp.float32(1.0))

            acc = lax.fori_loop(0, SAMPLE_NUM // LANES, chunk_body,
                                jnp.zeros((LANES,), jnp.float32))
            se = jnp.sum(acc)
            lane = jnp.int32(G) * g + qi - (g // G) * LANES
            pack = jnp.where(iota16 == _splat(lane), _splat(se), pack)
        se_v[pl.ds((g // G) * LANES, LANES)] = pack
        return pack

    start_ids(0, idsA_v, isemA)
    start_ids(1, idsB_v, isemB)
    wait_ids(idsA_v, isemA)
    fire_rows(idsA_v, bufA_v, semA)
    start_ids(2, idsA_v, isemA)
    wait_ids(idsB_v, isemB)
    fire_rows(idsB_v, bufB_v, semB)
    start_ids(3, idsB_v, isemB)

    def pipe_body(t, pack):
        gA = 2 * t
        gB = 2 * t + 1

        wait_rows(bufA_v, semA)
        pack = compute_group(gA, bufA_v, pack)

        @pl.when(gA + 2 < NGRP)
        def _():
            wait_ids(idsA_v, isemA)
            fire_rows(idsA_v, bufA_v, semA)

        @pl.when(gA + 4 < NGRP)
        def _():
            start_ids(gA + 4, idsA_v, isemA)

        wait_rows(bufB_v, semB)
        pack = compute_group(gB, bufB_v, pack)

        @pl.when(gB + 2 < NGRP)
        def _():
            wait_ids(idsB_v, isemB)
            fire_rows(idsB_v, bufB_v, semB)

        @pl.when(gB + 4 < NGRP)
        def _():
            start_ids(gB + 4, idsB_v, isemB)
        return pack

    lax.fori_loop(0, NGRP // 2, pipe_body, jnp.zeros((LANES,), jnp.float32))

    pltpu.sync_copy(se_v, out_hbm.at[pl.ds(wid * QPW, QPW)])
    pltpu.sync_copy(pos_v, out_hbm.at[pl.ds(NQ + wid * LANES, LANES)])


_sc_kernel = functools.partial(
    pl.kernel,
    mesh=plsc.VectorSubcoreMesh(core_axis_name="c", subcore_axis_name="s"),
    compiler_params=pltpu.CompilerParams(needs_layout_passes=False,
                                         use_tc_tiling_on_sc=False),
    out_type=jax.ShapeDtypeStruct((OUTLEN,), jnp.float32),
    scratch_types=[
        pltpu.VMEM((BPW * SEQ_LEN,), jnp.int32),
        pltpu.VMEM((BPW * SEQ_LEN, EMBED_DIM), jnp.float32),
        pltpu.VMEM((QPW * EMBED_DIM,), jnp.float32),
        pltpu.VMEM((QPW,), jnp.float32),
        pltpu.VMEM((4 * LANES,), jnp.float32),
        pltpu.VMEM((LANES,), jnp.float32),
        pltpu.VMEM((GROWS,), jnp.int32),
        pltpu.VMEM((GROWS,), jnp.int32),
        pltpu.VMEM((GROWS, EMBED_DIM), jnp.float32),
        pltpu.VMEM((GROWS, EMBED_DIM), jnp.float32),
        pltpu.SemaphoreType.DMA,
        pltpu.SemaphoreType.DMA,
        pltpu.SemaphoreType.DMA,
        pltpu.SemaphoreType.DMA,
        pltpu.SemaphoreType.DMA,
    ],
)(_sc_body)


def _tc_finish(x_ref, o_ref):
    x = x_ref[...]
    se = x[0:NQ // 128, :]
    pos = x[NQ // 128:, :]
    loss = (jnp.float32(NQ) + jnp.sum(jnp.log(se))
            - jnp.sum(pos) / jnp.float32(LANES))
    o_ref[...] = jnp.reshape(loss, (1, 1))


def kernel(seqs, item_embed):
    B, L = seqs.shape
    neg_key = jax.random.key(1234)
    keys = jax.vmap(lambda i: jax.random.fold_in(neg_key, i))(
        jnp.arange(1, L))
    samp = jax.vmap(
        lambda k: jax.random.randint(k, (B * SAMPLE_NUM,), 0, ITEM_NUM,
                                     dtype=jnp.int32))(keys)
    out = _sc_kernel(seqs.reshape(-1), item_embed, samp)
    loss = pl.pallas_call(
        _tc_finish,
        out_shape=jax.ShapeDtypeStruct((1, 1), jnp.float32),
    )(out.reshape(OUTLEN // 128, 128))
    return loss[0, 0]


# speedup vs baseline: 7.2079x; 1.0006x over previous
import functools

import jax
import jax.numpy as jnp
from jax import lax
from jax.experimental import pallas as pl
from jax.experimental.pallas import tpu as pltpu
from jax.experimental.pallas import tpu_sc as plsc

ITEM_NUM = 1000000
EMBED_DIM = 32
SAMPLE_NUM = 256
BATCH = 1024
SEQ_LEN = 20
NSTEP = SEQ_LEN - 1
NC, NS, LANES = 2, 16, 16
NW = NC * NS
BPW = BATCH // NW
QPW = BPW * NSTEP
NQ = BATCH * NSTEP
G = 4
GROWS = G * SAMPLE_NUM
NGRP = QPW // G
BG = BPW // G
OUTLEN = NQ + NW * LANES


def _splat(x):
    return jnp.broadcast_to(x, (LANES,))


def _rsqrt_vec(x):
    i = plsc.bitcast(x, jnp.int32)
    i = jnp.int32(0x5F3759DF) - lax.shift_right_logical(i, 1)
    y = plsc.bitcast(i, jnp.float32)
    for _ in range(3):
        y = y * (jnp.float32(1.5) - jnp.float32(0.5) * x * y * y)
    return y


def _sc_body(seqflat_hbm, table_hbm, samp_hbm, out_hbm,
             idxseq_v, rows_v, h_v, se_v, hh_v, pos_v,
             idsA_v, idsB_v, bufA_v, bufB_v,
             gsem, semA, semB, isemA, isemB):
    wid = lax.axis_index("s") * NC + lax.axis_index("c")
    base_b = wid * BPW
    iota16 = lax.iota(jnp.int32, LANES)

    pltpu.sync_copy(seqflat_hbm.at[pl.ds(base_b * SEQ_LEN, BPW * SEQ_LEN)],
                    idxseq_v)
    nseq = BPW * SEQ_LEN
    handles = []
    for c in range(nseq // 128):
        handles.append(pltpu.async_copy(
            table_hbm.at[idxseq_v.at[pl.ds(c * 128, 128)]],
            rows_v.at[pl.ds(c * 128, 128)], gsem))
    for h in handles:
        h.wait()

    def p1_body(bl, posacc):
        r0 = bl * SEQ_LEN
        s0 = jnp.zeros((LANES,), jnp.float32)
        s1 = jnp.zeros((LANES,), jnp.float32)
        for j in range(SEQ_LEN):
            e0 = rows_v[r0 + j, pl.ds(0, 16)]
            e1 = rows_v[r0 + j, pl.ds(16, 16)]
            inv = _rsqrt_vec(_splat(jnp.sum(e0 * e0) + jnp.sum(e1 * e1)))
            en0 = e0 * inv
            en1 = e1 * inv
            if j >= 1:
                hinv = _rsqrt_vec(_splat(jnp.sum(s0 * s0) + jnp.sum(s1 * s1)))
                h0 = s0 * hinv
                h1 = s1 * hinv
                q = bl * NSTEP + (j - 1)
                h_v[pl.ds(q * EMBED_DIM, 16)] = h0
                h_v[pl.ds(q * EMBED_DIM + 16, 16)] = h1
                posacc = posacc + jnp.sum(en0 * h0 + en1 * h1)
            s0 = s0 + en0
            s1 = s1 + en1
        return posacc

    posacc = lax.fori_loop(0, BPW, p1_body, jnp.float32(0.0))
    pos_v[...] = _splat(posacc)


    def start_ids(g, ids_v, isem):
        i = g // BG
        bg = g % BG
        r = pl.multiple_of((base_b + bg * G) * (SAMPLE_NUM // 128), 2)
        pltpu.async_copy(samp_hbm.at[i, pl.ds(r, GROWS // 128), :], ids_v,
                         isem)

    def wait_ids(ids_v, isem):
        pltpu.make_async_copy(samp_hbm.at[0, pl.ds(0, GROWS // 128), :],
                              ids_v, isem).wait()

    def fire_rows(ids_v, buf_v, sem):
        for k in range(GROWS // 128):
            pltpu.async_copy(table_hbm.at[ids_v.at[k, :]],
                             buf_v.at[pl.ds(k * 128, 128)], sem)

    def wait_rows(buf_v, sem):
        pltpu.make_async_copy(table_hbm.at[pl.ds(0, GROWS)], buf_v, sem).wait()

    def compute_group(g, buf_v, pack):
        i = g // BG
        bg = g % BG
        for qi in range(G):
            hrow = ((bg * G + qi) * NSTEP + i) * EMBED_DIM
            h0 = h_v[pl.ds(hrow, 16)]
            h1 = h_v[pl.ds(hrow + 16, 16)]
            hh_v[pl.ds(0, 16)] = h0
            hh_v[pl.ds(16, 16)] = h1
            hh_v[pl.ds(32, 16)] = h0
            hh_v[pl.ds(48, 16)] = h1
            rbase = qi * SAMPLE_NUM

            def chunk_body(c, acc):
                rows = iota16 + (rbase + c * LANES)
                acc_d = jnp.zeros((LANES,), jnp.float32)
                acc_s = jnp.zeros((LANES,), jnp.float32)
                diag = iota16
                for k in range(EMBED_DIM):
                    g_ = plsc.load_gather(buf_v, [rows, diag])
                    acc_d = acc_d + g_ * hh_v[pl.ds(k, 16)]
                    acc_s = acc_s + g_ * g_
                    diag = jnp.bitwise_and(diag + 1, EMBED_DIM - 1)
                return acc + jnp.exp(acc_d * _rsqrt_vec(acc_s)
                                     - jnp.float32(1.0))

            acc = lax.fori_loop(0, SAMPLE_NUM // LANES, chunk_body,
                                jnp.zeros((LANES,), jnp.float32))
            se = jnp.sum(acc)
            lane = jnp.int32(G) * g + qi - (g // G) * LANES
            pack = jnp.where(iota16 == _splat(lane), _splat(se), pack)
        se_v[pl.ds((g // G) * LANES, LANES)] = pack
        return pack

    start_ids(0, idsA_v, isemA)
    start_ids(1, idsB_v, isemB)
    wait_ids(idsA_v, isemA)
    fire_rows(idsA_v, bufA_v, semA)
    start_ids(2, idsA_v, isemA)
    wait_ids(idsB_v, isemB)
    fire_rows(idsB_v, bufB_v, semB)
    start_ids(3, idsB_v, isemB)

    def pipe_body(t, pack):
        gA = 2 * t
        gB = 2 * t + 1

        wait_rows(bufA_v, semA)
        pack = compute_group(gA, bufA_v, pack)

        @pl.when(gA + 2 < NGRP)
        def _():
            wait_ids(idsA_v, isemA)
            fire_rows(idsA_v, bufA_v, semA)

        @pl.when(gA + 4 < NGRP)
        def _():
            start_ids(gA + 4, idsA_v, isemA)

        wait_rows(bufB_v, semB)
        pack = compute_group(gB, bufB_v, pack)

        @pl.when(gB + 2 < NGRP)
        def _():
            wait_ids(idsB_v, isemB)
            fire_rows(idsB_v, bufB_v, semB)

        @pl.when(gB + 4 < NGRP)
        def _():
            start_ids(gB + 4, idsB_v, isemB)
        return pack

    lax.fori_loop(0, NGRP // 2, pipe_body, jnp.zeros((LANES,), jnp.float32))

    pltpu.sync_copy(se_v, out_hbm.at[pl.ds(wid * QPW, QPW)])
    pltpu.sync_copy(pos_v, out_hbm.at[pl.ds(NQ + wid * LANES, LANES)])


_sc_kernel = functools.partial(
    pl.kernel,
    mesh=plsc.VectorSubcoreMesh(core_axis_name="c", subcore_axis_name="s"),
    compiler_params=pltpu.CompilerParams(needs_layout_passes=False,
                                         use_tc_tiling_on_sc=False),
    out_type=jax.ShapeDtypeStruct((OUTLEN,), jnp.float32),
    scratch_types=[
        pltpu.VMEM((BPW * SEQ_LEN,), jnp.int32),
        pltpu.VMEM((BPW * SEQ_LEN, EMBED_DIM), jnp.float32),
        pltpu.VMEM((QPW * EMBED_DIM,), jnp.float32),
        pltpu.VMEM((QPW,), jnp.float32),
        pltpu.VMEM((4 * LANES,), jnp.float32),
        pltpu.VMEM((LANES,), jnp.float32),
        pltpu.VMEM((GROWS // 128, 128), jnp.int32),
        pltpu.VMEM((GROWS // 128, 128), jnp.int32),
        pltpu.VMEM((GROWS, EMBED_DIM), jnp.float32),
        pltpu.VMEM((GROWS, EMBED_DIM), jnp.float32),
        pltpu.SemaphoreType.DMA,
        pltpu.SemaphoreType.DMA,
        pltpu.SemaphoreType.DMA,
        pltpu.SemaphoreType.DMA,
        pltpu.SemaphoreType.DMA,
    ],
)(_sc_body)


def _tc_finish(x_ref, o_ref):
    x = x_ref[...]
    se = x[0:NQ // 128, :]
    pos = x[NQ // 128:, :]
    loss = (jnp.float32(NQ) + jnp.sum(jnp.log(se))
            - jnp.sum(pos) / jnp.float32(LANES))
    o_ref[...] = jnp.reshape(loss, (1, 1))


def kernel(seqs, item_embed):
    B, L = seqs.shape
    neg_key = jax.random.key(1234)
    keys = jax.vmap(lambda i: jax.random.fold_in(neg_key, i))(
        jnp.arange(1, L))
    samp = jax.vmap(
        lambda k: jax.random.randint(k, (B * SAMPLE_NUM,), 0, ITEM_NUM,
                                     dtype=jnp.int32))(keys)
    samp = samp.reshape(NSTEP, B * SAMPLE_NUM // 128, 128)
    out = _sc_kernel(seqs.reshape(-1), item_embed, samp)
    loss = pl.pallas_call(
        _tc_finish,
        out_shape=jax.ShapeDtypeStruct((1, 1), jnp.float32),
    )(out.reshape(OUTLEN // 128, 128))
    return loss[0, 0]


# speedup vs baseline: 7.2568x; 1.0068x over previous
import functools

import jax
import jax.numpy as jnp
from jax import lax
from jax.experimental import pallas as pl
from jax.experimental.pallas import tpu as pltpu
from jax.experimental.pallas import tpu_sc as plsc

ITEM_NUM = 1000000
EMBED_DIM = 32
SAMPLE_NUM = 256
BATCH = 1024
SEQ_LEN = 20
NSTEP = SEQ_LEN - 1
NC, NS, LANES = 2, 16, 16
NW = NC * NS
BPW = BATCH // NW
QPW = BPW * NSTEP
NQ = BATCH * NSTEP
G = 4
GROWS = G * SAMPLE_NUM
NGRP = QPW // G
BG = BPW // G
OUTLEN = NQ + NW * LANES


def _splat(x):
    return jnp.broadcast_to(x, (LANES,))


def _rsqrt_vec(x):
    i = plsc.bitcast(x, jnp.int32)
    i = jnp.int32(0x5F3759DF) - lax.shift_right_logical(i, 1)
    y = plsc.bitcast(i, jnp.float32)
    for _ in range(3):
        y = y * (jnp.float32(1.5) - jnp.float32(0.5) * x * y * y)
    return y


def _sc_body(seqflat_hbm, table_hbm, samp_hbm, out_hbm,
             idxseq_v, rows_v, h_v, se_v, hh_v, pos_v,
             idsA_v, idsB_v, bufA_v, bufB_v,
             gsem, semA, semB, isemA, isemB):
    wid = lax.axis_index("s") * NC + lax.axis_index("c")
    base_b = wid * BPW
    iota16 = lax.iota(jnp.int32, LANES)

    pltpu.sync_copy(seqflat_hbm.at[pl.ds(base_b * SEQ_LEN, BPW * SEQ_LEN)],
                    idxseq_v)
    nseq = BPW * SEQ_LEN
    handles = []
    for c in range(nseq // 128):
        handles.append(pltpu.async_copy(
            table_hbm.at[idxseq_v.at[pl.ds(c * 128, 128)]],
            rows_v.at[pl.ds(c * 128, 128)], gsem))
    for h in handles:
        h.wait()

    def p1_body(bl, posacc):
        r0 = bl * SEQ_LEN
        s0 = jnp.zeros((LANES,), jnp.float32)
        s1 = jnp.zeros((LANES,), jnp.float32)
        for j in range(SEQ_LEN):
            e0 = rows_v[r0 + j, pl.ds(0, 16)]
            e1 = rows_v[r0 + j, pl.ds(16, 16)]
            inv = _rsqrt_vec(_splat(jnp.sum(e0 * e0) + jnp.sum(e1 * e1)))
            en0 = e0 * inv
            en1 = e1 * inv
            if j >= 1:
                hinv = _rsqrt_vec(_splat(jnp.sum(s0 * s0) + jnp.sum(s1 * s1)))
                h0 = s0 * hinv
                h1 = s1 * hinv
                q = bl * NSTEP + (j - 1)
                h_v[pl.ds(q * EMBED_DIM, 16)] = h0
                h_v[pl.ds(q * EMBED_DIM + 16, 16)] = h1
                posacc = posacc + jnp.sum(en0 * h0 + en1 * h1)
            s0 = s0 + en0
            s1 = s1 + en1
        return posacc

    posacc = lax.fori_loop(0, BPW, p1_body, jnp.float32(0.0))
    pos_v[...] = _splat(posacc)


    def start_ids(g, ids_v, isem):
        i = g // BG
        bg = g % BG
        r = pl.multiple_of((base_b + bg * G) * (SAMPLE_NUM // 128), 2)
        pltpu.async_copy(samp_hbm.at[i, pl.ds(r, GROWS // 128), :], ids_v,
                         isem)

    def wait_ids(ids_v, isem):
        pltpu.make_async_copy(samp_hbm.at[0, pl.ds(0, GROWS // 128), :],
                              ids_v, isem).wait()

    def fire_rows(ids_v, buf_v, sem):
        for k in range(GROWS // 128):
            pltpu.async_copy(table_hbm.at[ids_v.at[k, :]],
                             buf_v.at[pl.ds(k * 128, 128)], sem)

    def wait_rows(buf_v, sem):
        pltpu.make_async_copy(table_hbm.at[pl.ds(0, GROWS)], buf_v, sem).wait()

    def compute_group(g, buf_v, pack):
        i = g // BG
        bg = g % BG
        for qi in range(G):
            hrow = ((bg * G + qi) * NSTEP + i) * EMBED_DIM
            h0 = h_v[pl.ds(hrow, 16)]
            h1 = h_v[pl.ds(hrow + 16, 16)]
            hh_v[pl.ds(0, 16)] = h0
            hh_v[pl.ds(16, 16)] = h1
            hh_v[pl.ds(32, 16)] = h0
            hh_v[pl.ds(48, 16)] = h1
            rbase = qi * SAMPLE_NUM

            def chunk_body(c, acc):
                rows = iota16 + (rbase + c * LANES)
                acc_d = jnp.zeros((LANES,), jnp.float32)
                acc_s = jnp.zeros((LANES,), jnp.float32)
                diag = iota16
                for k in range(EMBED_DIM):
                    g_ = plsc.load_gather(buf_v, [rows, diag])
                    acc_d = acc_d + g_ * hh_v[pl.ds(k, 16)]
                    acc_s = acc_s + g_ * g_
                    diag = jnp.bitwise_and(diag + 1, EMBED_DIM - 1)
                return acc + jnp.exp(acc_d * _rsqrt_vec(acc_s)
                                     - jnp.float32(1.0))

            acc = lax.fori_loop(0, SAMPLE_NUM // LANES, chunk_body,
                                jnp.zeros((LANES,), jnp.float32))
            se = jnp.sum(acc)
            lane = jnp.int32(G) * g + qi - (g // G) * LANES
            pack = jnp.where(iota16 == _splat(lane), _splat(se), pack)
        se_v[pl.ds((g // G) * LANES, LANES)] = pack
        return pack

    start_ids(0, idsA_v, isemA)
    start_ids(1, idsB_v, isemB)
    wait_ids(idsA_v, isemA)
    fire_rows(idsA_v, bufA_v, semA)
    start_ids(2, idsA_v, isemA)
    wait_ids(idsB_v, isemB)
    fire_rows(idsB_v, bufB_v, semB)
    start_ids(3, idsB_v, isemB)

    def pipe_body(t, pack):
        gA = 2 * t
        gB = 2 * t + 1

        wait_rows(bufA_v, semA)
        pack = compute_group(gA, bufA_v, pack)

        @pl.when(gA + 2 < NGRP)
        def _():
            wait_ids(idsA_v, isemA)
            fire_rows(idsA_v, bufA_v, semA)

        @pl.when(gA + 4 < NGRP)
        def _():
            start_ids(gA + 4, idsA_v, isemA)

        wait_rows(bufB_v, semB)
        pack = compute_group(gB, bufB_v, pack)

        @pl.when(gB + 2 < NGRP)
        def _():
            wait_ids(idsB_v, isemB)
            fire_rows(idsB_v, bufB_v, semB)

        @pl.when(gB + 4 < NGRP)
        def _():
            start_ids(gB + 4, idsB_v, isemB)
        return pack

    lax.fori_loop(0, NGRP // 2, pipe_body, jnp.zeros((LANES,), jnp.float32))

    pltpu.sync_copy(se_v, out_hbm.at[pl.ds(wid * QPW, QPW)])
    pltpu.sync_copy(pos_v, out_hbm.at[pl.ds(NQ + wid * LANES, LANES)])


_sc_kernel = functools.partial(
    pl.kernel,
    mesh=plsc.VectorSubcoreMesh(core_axis_name="c", subcore_axis_name="s"),
    compiler_params=pltpu.CompilerParams(needs_layout_passes=False,
                                         use_tc_tiling_on_sc=False),
    out_type=jax.ShapeDtypeStruct((OUTLEN,), jnp.float32),
    scratch_types=[
        pltpu.VMEM((BPW * SEQ_LEN,), jnp.int32),
        pltpu.VMEM((BPW * SEQ_LEN, EMBED_DIM), jnp.float32),
        pltpu.VMEM((QPW * EMBED_DIM,), jnp.float32),
        pltpu.VMEM((QPW,), jnp.float32),
        pltpu.VMEM((4 * LANES,), jnp.float32),
        pltpu.VMEM((LANES,), jnp.float32),
        pltpu.VMEM((GROWS // 128, 128), jnp.int32),
        pltpu.VMEM((GROWS // 128, 128), jnp.int32),
        pltpu.VMEM((GROWS, EMBED_DIM), jnp.float32),
        pltpu.VMEM((GROWS, EMBED_DIM), jnp.float32),
        pltpu.SemaphoreType.DMA,
        pltpu.SemaphoreType.DMA,
        pltpu.SemaphoreType.DMA,
        pltpu.SemaphoreType.DMA,
        pltpu.SemaphoreType.DMA,
    ],
)(_sc_body)


def _tc_finish(x_ref, o_ref):
    x = x_ref[...]
    se = x[0:NQ // 128, :]
    pos = x[NQ // 128:, :]
    loss = (jnp.float32(NQ) + jnp.sum(jnp.log(se))
            - jnp.sum(pos) / jnp.float32(LANES))
    o_ref[...] = jnp.reshape(loss, (1, 1))


def kernel(seqs, item_embed):
    B, L = seqs.shape
    neg_key = jax.random.key(1234)
    keys = jax.vmap(lambda i: jax.random.fold_in(neg_key, i))(
        jnp.arange(1, L))
    samp = jax.vmap(
        lambda k: jax.random.randint(k, (B * SAMPLE_NUM // 128, 128), 0,
                                     ITEM_NUM, dtype=jnp.int32))(keys)
    out = _sc_kernel(seqs.reshape(-1), item_embed, samp)
    loss = pl.pallas_call(
        _tc_finish,
        out_shape=jax.ShapeDtypeStruct((1, 1), jnp.float32),
    )(out.reshape(OUTLEN // 128, 128))
    return loss[0, 0]
